# baseline (speedup 1.0000x reference)
import functools

import jax
import jax.numpy as jnp
from jax.experimental import pallas as pl
from jax.experimental.pallas import tpu as pltpu


def _fused_kernel(feat_ref, w_ref, b_ref, adj_a_ref, adj_b_ref,
                  diag_a_ref, diag_b_ref, out_ref, pcat_ref, x_ref,
                  *, g, bm, d):
    i = pl.program_id(0)

    @pl.when(i == 0)
    def _init():
        p = jnp.dot(feat_ref[...], w_ref[...].T,
                    preferred_element_type=jnp.float32) + b_ref[...]
        pcat_ref[:, :d] = p.astype(jnp.bfloat16)
        pcat_ref[:, d:] = (p * p).astype(jnp.bfloat16)

    @pl.when(i < g)
    def _phase_adj():
        for half, ref in ((0, adj_a_ref), (1, adj_b_ref)):
            sq = jnp.dot(ref[...].astype(jnp.bfloat16), pcat_ref[...],
                         preferred_element_type=jnp.float32)
            s = sq[:, :d]
            q = sq[:, d:]
            x_ref[pl.ds((2 * i + half) * bm, bm), :] = (
                0.5 * (s * s - q)).astype(jnp.bfloat16)

    @pl.when(i >= g)
    def _phase_diag():
        for half, ref in ((0, diag_a_ref), (1, diag_b_ref)):
            out_ref[pl.ds(half * bm, bm), :] = jnp.dot(
                ref[...].astype(jnp.bfloat16), x_ref[...],
                preferred_element_type=jnp.float32)


def kernel(feat, adj_loop, diag_mat, W, b):
    n, _ = feat.shape
    d = W.shape[0]
    bm = 40 if n % 80 == 0 else n
    g = n // (2 * bm)

    adj_a = lambda i: (2 * jnp.minimum(i, g - 1), 0)
    adj_b = lambda i: (2 * jnp.minimum(i, g - 1) + 1, 0)
    diag_a = lambda i: (2 * jnp.maximum(i - g, 0), 0)
    diag_b = lambda i: (2 * jnp.maximum(i - g, 0) + 1, 0)

    return pl.pallas_call(
        functools.partial(_fused_kernel, g=g, bm=bm, d=d),
        grid=(2 * g,),
        in_specs=[
            pl.BlockSpec((n, feat.shape[1]), lambda i: (0, 0)),
            pl.BlockSpec((d, W.shape[1]), lambda i: (0, 0)),
            pl.BlockSpec((1, d), lambda i: (0, 0)),
            pl.BlockSpec((bm, n), adj_a),
            pl.BlockSpec((bm, n), adj_b),
            pl.BlockSpec((bm, n), diag_a),
            pl.BlockSpec((bm, n), diag_b),
        ],
        out_specs=pl.BlockSpec((2 * bm, d), lambda i: (jnp.maximum(i - g, 0), 0)),
        out_shape=jax.ShapeDtypeStruct((n, d), jnp.float32),
        scratch_shapes=[
            pltpu.VMEM((n, 2 * d), jnp.bfloat16),
            pltpu.VMEM((n, d), jnp.bfloat16),
        ],
    )(feat, W, b.reshape(1, d), adj_loop, adj_loop, diag_mat, diag_mat)


# speedup vs baseline: 1.7050x; 1.7050x over previous
import functools

import jax
import jax.numpy as jnp
from jax.experimental import pallas as pl
from jax.experimental.pallas import tpu as pltpu


def _fused_kernel(feat_ref, w_ref, b_ref, adj_ref, diag_ref, out_ref,
                  pcat_ref, x_ref, *, g, bm, d):
    i = pl.program_id(0)

    @pl.when(i == 0)
    def _init():
        p = jnp.dot(feat_ref[...], w_ref[...].T,
                    preferred_element_type=jnp.float32) + b_ref[...]
        pcat_ref[:, :d] = p.astype(jnp.bfloat16)
        pcat_ref[:, d:] = (p * p).astype(jnp.bfloat16)

    @pl.when(i < g)
    def _phase_adj():
        sq = jnp.dot(adj_ref[...].astype(jnp.bfloat16), pcat_ref[...],
                     preferred_element_type=jnp.float32)
        s = sq[:, :d]
        q = sq[:, d:]
        x_ref[pl.ds(i * bm, bm), :] = (0.5 * (s * s - q)).astype(jnp.bfloat16)

    @pl.when(i >= g)
    def _phase_diag():
        out_ref[...] = jnp.dot(diag_ref[...].astype(jnp.bfloat16), x_ref[...],
                               preferred_element_type=jnp.float32)


def kernel(feat, adj_loop, diag_mat, W, b):
    n, _ = feat.shape
    d = W.shape[0]
    bm = 200 if n % 200 == 0 else n
    g = n // bm

    return pl.pallas_call(
        functools.partial(_fused_kernel, g=g, bm=bm, d=d),
        grid=(2 * g,),
        in_specs=[
            pl.BlockSpec((n, feat.shape[1]), lambda i: (0, 0)),
            pl.BlockSpec((d, W.shape[1]), lambda i: (0, 0)),
            pl.BlockSpec((1, d), lambda i: (0, 0)),
            pl.BlockSpec((bm, n), lambda i: (jnp.minimum(i, g - 1), 0)),
            pl.BlockSpec((bm, n), lambda i: (jnp.maximum(i - g, 0), 0)),
        ],
        out_specs=pl.BlockSpec((bm, d), lambda i: (jnp.maximum(i - g, 0), 0)),
        out_shape=jax.ShapeDtypeStruct((n, d), jnp.float32),
        scratch_shapes=[
            pltpu.VMEM((n, 2 * d), jnp.bfloat16),
            pltpu.VMEM((n, d), jnp.bfloat16),
        ],
    )(feat, W, b.reshape(1, d), adj_loop, diag_mat)


# speedup vs baseline: 1.7581x; 1.0312x over previous
import functools

import jax
import jax.numpy as jnp
from jax.experimental import pallas as pl
from jax.experimental.pallas import tpu as pltpu

_DEPTH = 2


def _fused_kernel(feat_ref, w_ref, b_ref, adj_hbm, diag_hbm, out_ref,
                  pcat_ref, x_ref, buf0_ref, buf1_ref, sem, *, g, bm, d):
    i = pl.program_id(0)
    bufs = (buf0_ref, buf1_ref)

    def issue(j, k):
        @pl.when(j < g)
        def _():
            pltpu.make_async_copy(adj_hbm.at[pl.ds(j * bm, bm), :],
                                  bufs[k], sem.at[k]).start()

        @pl.when(jnp.logical_and(j >= g, j < 2 * g))
        def _():
            pltpu.make_async_copy(diag_hbm.at[pl.ds((j - g) * bm, bm), :],
                                  bufs[k], sem.at[k]).start()

    @pl.when(i == 0)
    def _prologue():
        for k in range(_DEPTH):
            issue(jnp.int32(k), k)
        p = jnp.dot(feat_ref[...], w_ref[...].T,
                    preferred_element_type=jnp.float32) + b_ref[...]
        pcat_ref[:, :d] = p
        pcat_ref[:, d:] = p * p

    slot = jax.lax.rem(i, _DEPTH)

    def step_body(k):
        pltpu.make_async_copy(adj_hbm.at[pl.ds(0, bm), :],
                              bufs[k], sem.at[k]).wait()

        @pl.when(i < g)
        def _phase_adj():
            sq = jnp.dot(bufs[k][...], pcat_ref[...],
                         preferred_element_type=jnp.float32)
            s = sq[:, :d]
            q = sq[:, d:]
            x_ref[pl.ds(i * bm, bm), :] = 0.5 * (s * s - q)

        @pl.when(i >= g)
        def _phase_diag():
            out_ref[...] = jnp.dot(bufs[k][...], x_ref[...],
                                   preferred_element_type=jnp.float32)

        issue(i + _DEPTH, k)

    for k in range(_DEPTH):
        pl.when(slot == k)(functools.partial(step_body, k))


def kernel(feat, adj_loop, diag_mat, W, b):
    n, _ = feat.shape
    d = W.shape[0]
    bm = 400 if n % 400 == 0 else n
    g = n // bm

    return pl.pallas_call(
        functools.partial(_fused_kernel, g=g, bm=bm, d=d),
        grid=(2 * g,),
        in_specs=[
            pl.BlockSpec((n, feat.shape[1]), lambda i: (0, 0)),
            pl.BlockSpec((d, W.shape[1]), lambda i: (0, 0)),
            pl.BlockSpec((1, d), lambda i: (0, 0)),
            pl.BlockSpec(memory_space=pltpu.MemorySpace.HBM),
            pl.BlockSpec(memory_space=pltpu.MemorySpace.HBM),
        ],
        out_specs=pl.BlockSpec((bm, d), lambda i: (jnp.maximum(i - g, 0), 0)),
        out_shape=jax.ShapeDtypeStruct((n, d), jnp.float32),
        scratch_shapes=[
            pltpu.VMEM((n, 2 * d), jnp.float32),
            pltpu.VMEM((n, d), jnp.float32),
            pltpu.VMEM((bm, n), jnp.float32),
            pltpu.VMEM((bm, n), jnp.float32),
            pltpu.SemaphoreType.DMA((_DEPTH,)),
        ],
    )(feat, W, b.reshape(1, d), adj_loop, diag_mat)
